# baseline (device time: 30070 ns/iter reference)
import jax
import jax.numpy as jnp
from jax import lax
from jax.experimental import pallas as pl
from jax.experimental.pallas import tpu as pltpu


def kernel(x, router, W1, W2):
    t_loc, d = x.shape
    f = W1.shape[2]

    def body(x_ref, r_ref, w1_ref, w2_ref, out_ref,
             xrecv, rrecv, part, precv, send_sems, recv_sems):
        my_x = lax.axis_index("x")
        my_y = lax.axis_index("y")
        my_z = lax.axis_index("z")
        peer = (my_x, 1 - my_y, my_z)

        barrier_sem = pltpu.get_barrier_semaphore()
        pl.semaphore_signal(barrier_sem, inc=1, device_id=peer,
                            device_id_type=pl.DeviceIdType.MESH)
        pl.semaphore_wait(barrier_sem, 1)

        rdma_x = pltpu.make_async_remote_copy(
            src_ref=x_ref, dst_ref=xrecv,
            send_sem=send_sems.at[0], recv_sem=recv_sems.at[0],
            device_id=peer, device_id_type=pl.DeviceIdType.MESH)
        rdma_r = pltpu.make_async_remote_copy(
            src_ref=r_ref, dst_ref=rrecv,
            send_sem=send_sems.at[1], recv_sem=recv_sems.at[1],
            device_id=peer, device_id_type=pl.DeviceIdType.MESH)
        rdma_x.start()
        rdma_r.start()
        rdma_x.wait()
        rdma_r.wait()

        r_loc = r_ref[...]
        r_rem = rrecv[...]

        def partial_for(xg):
            gl = jnp.dot(xg, r_loc, preferred_element_type=jnp.float32)
            gr = jnp.dot(xg, r_rem, preferred_element_type=jnp.float32)
            g0, g1 = gl[:, 0:1], gl[:, 1:2]
            g2, g3 = gr[:, 0:1], gr[:, 1:2]
            m1 = jnp.maximum(jnp.maximum(g0, g1), jnp.maximum(g2, g3))
            big_neg = jnp.float32(-1e30)
            m2 = jnp.maximum(
                jnp.maximum(jnp.where(g0 == m1, big_neg, g0),
                            jnp.where(g1 == m1, big_neg, g1)),
                jnp.maximum(jnp.where(g2 == m1, big_neg, g2),
                            jnp.where(g3 == m1, big_neg, g3)))
            denom = 1.0 + jnp.exp(m2 - m1)

            def wcol(gc):
                return jnp.where(gc >= m2, jnp.exp(gc - m1), 0.0) / denom

            w0, w1 = wcol(g0), wcol(g1)
            h0 = jnp.maximum(
                jnp.dot(xg, w1_ref[0], preferred_element_type=jnp.float32), 0.0)
            y0 = jnp.dot(h0, w2_ref[0], preferred_element_type=jnp.float32)
            h1 = jnp.maximum(
                jnp.dot(xg, w1_ref[1], preferred_element_type=jnp.float32), 0.0)
            y1 = jnp.dot(h1, w2_ref[1], preferred_element_type=jnp.float32)
            return y0 * w0 + y1 * w1

        part[0, :, :] = partial_for(x_ref[...])
        part[1, :, :] = partial_for(xrecv[...])

        rdma_p = pltpu.make_async_remote_copy(
            src_ref=part.at[1], dst_ref=precv,
            send_sem=send_sems.at[2], recv_sem=recv_sems.at[2],
            device_id=peer, device_id_type=pl.DeviceIdType.MESH)
        rdma_p.start()
        rdma_p.wait()

        out_ref[...] = part[0, :, :] + precv[...]

    return pl.pallas_call(
        body,
        out_shape=jax.ShapeDtypeStruct((t_loc, d), jnp.float32),
        in_specs=[pl.BlockSpec(memory_space=pltpu.VMEM)] * 4,
        out_specs=pl.BlockSpec(memory_space=pltpu.VMEM),
        scratch_shapes=[
            pltpu.VMEM((t_loc, d), jnp.float32),
            pltpu.VMEM(router.shape, jnp.float32),
            pltpu.VMEM((2, t_loc, d), jnp.float32),
            pltpu.VMEM((t_loc, d), jnp.float32),
            pltpu.SemaphoreType.DMA((3,)),
            pltpu.SemaphoreType.DMA((3,)),
        ],
        compiler_params=pltpu.CompilerParams(collective_id=0),
    )(x, router, W1, W2)


# device time: 28619 ns/iter; 1.0507x vs baseline; 1.0507x over previous
import jax
import jax.numpy as jnp
from jax import lax
from jax.experimental import pallas as pl
from jax.experimental.pallas import tpu as pltpu


def kernel(x, router, W1, W2):
    t_loc, d = x.shape
    f = W1.shape[2]

    def body(x_ref, r_ref, w1_ref, w2_ref, out_ref,
             xrecv, rrecv, part, precv, send_sems, recv_sems):
        my_x = lax.axis_index("x")
        my_y = lax.axis_index("y")
        my_z = lax.axis_index("z")
        peer = (my_x, 1 - my_y, my_z)

        barrier_sem = pltpu.get_barrier_semaphore()
        pl.semaphore_signal(barrier_sem, inc=1, device_id=peer,
                            device_id_type=pl.DeviceIdType.MESH)
        pl.semaphore_wait(barrier_sem, 1)

        rdma_x = pltpu.make_async_remote_copy(
            src_ref=x_ref, dst_ref=xrecv,
            send_sem=send_sems.at[0], recv_sem=recv_sems.at[0],
            device_id=peer, device_id_type=pl.DeviceIdType.MESH)
        rdma_r = pltpu.make_async_remote_copy(
            src_ref=r_ref, dst_ref=rrecv,
            send_sem=send_sems.at[1], recv_sem=recv_sems.at[1],
            device_id=peer, device_id_type=pl.DeviceIdType.MESH)
        rdma_x.start()
        rdma_r.start()

        bf16 = jnp.bfloat16
        w10 = w1_ref[0].astype(bf16)
        w11 = w1_ref[1].astype(bf16)
        w20 = w2_ref[0].astype(bf16)
        w21 = w2_ref[1].astype(bf16)

        def ffn(xg):
            xb = xg.astype(bf16)
            h0 = jnp.maximum(
                jnp.dot(xb, w10, preferred_element_type=jnp.float32), 0.0)
            y0 = jnp.dot(h0.astype(bf16), w20,
                         preferred_element_type=jnp.float32)
            h1 = jnp.maximum(
                jnp.dot(xb, w11, preferred_element_type=jnp.float32), 0.0)
            y1 = jnp.dot(h1.astype(bf16), w21,
                         preferred_element_type=jnp.float32)
            return y0, y1

        def local_weights(xg, r_loc, r_rem):
            gl = jnp.dot(xg, r_loc, preferred_element_type=jnp.float32)
            gr = jnp.dot(xg, r_rem, preferred_element_type=jnp.float32)
            g0, g1 = gl[:, 0:1], gl[:, 1:2]
            g2, g3 = gr[:, 0:1], gr[:, 1:2]
            m1 = jnp.maximum(jnp.maximum(g0, g1), jnp.maximum(g2, g3))
            big_neg = jnp.float32(-1e30)
            m2 = jnp.maximum(
                jnp.maximum(jnp.where(g0 == m1, big_neg, g0),
                            jnp.where(g1 == m1, big_neg, g1)),
                jnp.maximum(jnp.where(g2 == m1, big_neg, g2),
                            jnp.where(g3 == m1, big_neg, g3)))
            denom = 1.0 + jnp.exp(m2 - m1)

            def wcol(gc):
                return jnp.where(gc >= m2, jnp.exp(gc - m1), 0.0) / denom

            return wcol(g0), wcol(g1)

        y0m, y1m = ffn(x_ref[...])

        rdma_x.wait()
        rdma_r.wait()

        xp = xrecv[...]
        w0p, w1p = local_weights(xp, r_ref[...], rrecv[...])
        y0p, y1p = ffn(xp)
        part[0, :, :] = y0p * w0p + y1p * w1p
        rdma_p = pltpu.make_async_remote_copy(
            src_ref=part.at[0], dst_ref=precv,
            send_sem=send_sems.at[2], recv_sem=recv_sems.at[2],
            device_id=peer, device_id_type=pl.DeviceIdType.MESH)
        rdma_p.start()

        w0m, w1m = local_weights(x_ref[...], r_ref[...], rrecv[...])
        mine = y0m * w0m + y1m * w1m

        rdma_p.wait()
        out_ref[...] = mine + precv[...]

    return pl.pallas_call(
        body,
        out_shape=jax.ShapeDtypeStruct((t_loc, d), jnp.float32),
        in_specs=[pl.BlockSpec(memory_space=pltpu.VMEM)] * 4,
        out_specs=pl.BlockSpec(memory_space=pltpu.VMEM),
        scratch_shapes=[
            pltpu.VMEM((t_loc, d), jnp.float32),
            pltpu.VMEM(router.shape, jnp.float32),
            pltpu.VMEM((1, t_loc, d), jnp.float32),
            pltpu.VMEM((t_loc, d), jnp.float32),
            pltpu.SemaphoreType.DMA((3,)),
            pltpu.SemaphoreType.DMA((3,)),
        ],
        compiler_params=pltpu.CompilerParams(collective_id=0),
    )(x, router, W1, W2)


# device time: 23696 ns/iter; 1.2690x vs baseline; 1.2078x over previous
import jax
import jax.numpy as jnp
from jax import lax
from jax.experimental import pallas as pl
from jax.experimental.pallas import tpu as pltpu


def kernel(x, router, W1, W2):
    t_loc, d = x.shape

    def body(x_ref, r_ref, w1_hbm, w2_hbm, out_ref,
             xsend, xrecv, wsend, wrecv, rrecv, psend, precv,
             w1v, w2v, send_sems, recv_sems, wsems):
        my_x = lax.axis_index("x")
        my_y = lax.axis_index("y")
        my_z = lax.axis_index("z")
        peer = (my_x, 1 - my_y, my_z)

        cp1 = pltpu.make_async_copy(w1_hbm, w1v, wsems.at[0])
        cp2 = pltpu.make_async_copy(w2_hbm, w2v, wsems.at[1])
        cp1.start()
        cp2.start()

        barrier_sem = pltpu.get_barrier_semaphore()
        pl.semaphore_signal(barrier_sem, inc=1, device_id=peer,
                            device_id_type=pl.DeviceIdType.MESH)
        pl.semaphore_wait(barrier_sem, 1)

        rdma_r = pltpu.make_async_remote_copy(
            src_ref=r_ref, dst_ref=rrecv,
            send_sem=send_sems.at[0], recv_sem=recv_sems.at[0],
            device_id=peer, device_id_type=pl.DeviceIdType.MESH)
        rdma_r.start()
        xsend[...] = x_ref[...].astype(jnp.bfloat16)
        rdma_x = pltpu.make_async_remote_copy(
            src_ref=xsend, dst_ref=xrecv,
            send_sem=send_sems.at[1], recv_sem=recv_sems.at[1],
            device_id=peer, device_id_type=pl.DeviceIdType.MESH)
        rdma_x.start()

        xm = x_ref[...]
        gl = jnp.dot(xm, r_ref[...], preferred_element_type=jnp.float32)
        rdma_r.wait()
        gr = jnp.dot(xm, rrecv[...], preferred_element_type=jnp.float32)
        g0, g1 = gl[:, 0:1], gl[:, 1:2]
        g2, g3 = gr[:, 0:1], gr[:, 1:2]
        m1 = jnp.maximum(jnp.maximum(g0, g1), jnp.maximum(g2, g3))
        big_neg = jnp.float32(-1e30)
        m2 = jnp.maximum(
            jnp.maximum(jnp.where(g0 == m1, big_neg, g0),
                        jnp.where(g1 == m1, big_neg, g1)),
            jnp.maximum(jnp.where(g2 == m1, big_neg, g2),
                        jnp.where(g3 == m1, big_neg, g3)))
        denom = 1.0 + jnp.exp(m2 - m1)

        def wcol(gc):
            return jnp.where(gc >= m2, jnp.exp(gc - m1), 0.0) / denom

        w0m, w1m = wcol(g0), wcol(g1)
        wsend[:, 0:1] = wcol(g2)
        wsend[:, 1:2] = wcol(g3)
        rdma_w = pltpu.make_async_remote_copy(
            src_ref=wsend, dst_ref=wrecv,
            send_sem=send_sems.at[2], recv_sem=recv_sems.at[2],
            device_id=peer, device_id_type=pl.DeviceIdType.MESH)
        rdma_w.start()

        cp1.wait()
        cp2.wait()
        bf16 = jnp.bfloat16
        w1b0, w1b1 = w1v[0].astype(bf16), w1v[1].astype(bf16)
        w2b0, w2b1 = w2v[0].astype(bf16), w2v[1].astype(bf16)

        def ffn(xb):
            h0 = jnp.maximum(
                jnp.dot(xb, w1b0, preferred_element_type=jnp.float32), 0.0)
            y0 = jnp.dot(h0.astype(bf16), w2b0,
                         preferred_element_type=jnp.float32)
            h1 = jnp.maximum(
                jnp.dot(xb, w1b1, preferred_element_type=jnp.float32), 0.0)
            y1 = jnp.dot(h1.astype(bf16), w2b1,
                         preferred_element_type=jnp.float32)
            return y0, y1

        y0m, y1m = ffn(xsend[...])
        mine = y0m * w0m + y1m * w1m

        rdma_x.wait()
        y0p, y1p = ffn(xrecv[...])
        rdma_w.wait()
        psend[...] = (y0p * wrecv[:, 0:1]
                      + y1p * wrecv[:, 1:2]).astype(jnp.bfloat16)
        rdma_p = pltpu.make_async_remote_copy(
            src_ref=psend, dst_ref=precv,
            send_sem=send_sems.at[3], recv_sem=recv_sems.at[3],
            device_id=peer, device_id_type=pl.DeviceIdType.MESH)
        rdma_p.start()
        rdma_p.wait()

        out_ref[...] = mine + precv[...].astype(jnp.float32)

    return pl.pallas_call(
        body,
        out_shape=jax.ShapeDtypeStruct((t_loc, d), jnp.float32),
        in_specs=[
            pl.BlockSpec(memory_space=pltpu.VMEM),
            pl.BlockSpec(memory_space=pltpu.VMEM),
            pl.BlockSpec(memory_space=pl.ANY),
            pl.BlockSpec(memory_space=pl.ANY),
        ],
        out_specs=pl.BlockSpec(memory_space=pltpu.VMEM),
        scratch_shapes=[
            pltpu.VMEM((t_loc, d), jnp.bfloat16),
            pltpu.VMEM((t_loc, d), jnp.bfloat16),
            pltpu.VMEM((t_loc, 2), jnp.float32),
            pltpu.VMEM((t_loc, 2), jnp.float32),
            pltpu.VMEM(router.shape, jnp.float32),
            pltpu.VMEM((t_loc, d), jnp.bfloat16),
            pltpu.VMEM((t_loc, d), jnp.bfloat16),
            pltpu.VMEM(W1.shape, jnp.float32),
            pltpu.VMEM(W2.shape, jnp.float32),
            pltpu.SemaphoreType.DMA((4,)),
            pltpu.SemaphoreType.DMA((4,)),
            pltpu.SemaphoreType.DMA((2,)),
        ],
        compiler_params=pltpu.CompilerParams(collective_id=0),
    )(x, router, W1, W2)
